# baseline (speedup 1.0000x reference)
import jax
import jax.numpy as jnp
from jax import lax
from jax.experimental import pallas as pl
from jax.experimental.pallas import tpu as pltpu
from jax.experimental.pallas import tpu_sc as plsc

BATCH = 16384
VOCAB = 100000
DIM = 64
LANES = 16

NUM_CORES = 2
NUM_SUBCORES = 16
NW = NUM_CORES * NUM_SUBCORES
COLS_PER_W = (BATCH // 128) // NW
TILE_WORDS = 8 * 128
COL_WORDS = DIM * 128


def _body(x_hbm, idx_hbm, emb_hbm, out_hbm, idx_v, g_v, x_v, gsem):
    wid = lax.axis_index("s") * NUM_CORES + lax.axis_index("c")

    descs = []
    for k in range(COLS_PER_W):
        t1 = wid * COLS_PER_W + k
        pltpu.sync_copy(idx_hbm.at[pl.ds(t1 * 128, 128)], idx_v.at[k])
        descs.append(
            pltpu.async_copy(emb_hbm.at[idx_v.at[k]], g_v.at[k], gsem)
        )

    for k in range(COLS_PER_W):
        t1 = wid * COLS_PER_W + k
        for t0 in range(8):
            pltpu.sync_copy(
                x_hbm.at[pl.ds((t0 * 128 + t1) * TILE_WORDS, TILE_WORDS)],
                x_v.at[k, pl.ds(t0 * TILE_WORDS, TILE_WORDS)],
            )
    for d in descs:
        d.wait()

    lane_iota = lax.iota(jnp.int32, LANES)
    for k in range(COLS_PER_W):
        g_k = g_v.at[k]

        def add_dim(d, carry, g_k=g_k, k=k):
            dcol = jnp.full((LANES,), d, dtype=jnp.int32)
            for lc in range(128 // LANES):
                rows = lane_iota + (lc * LANES)
                col = plsc.load_gather(g_k, [rows, dcol])
                sl = pl.ds(d * 128 + lc * LANES, LANES)
                x_v[k, sl] = x_v[k, sl] + col
            return carry

        lax.fori_loop(0, DIM, add_dim, 0)

    for k in range(COLS_PER_W):
        t1 = wid * COLS_PER_W + k
        for t0 in range(8):
            pltpu.sync_copy(
                x_v.at[k, pl.ds(t0 * TILE_WORDS, TILE_WORDS)],
                out_hbm.at[pl.ds((t0 * 128 + t1) * TILE_WORDS, TILE_WORDS)],
            )


@jax.jit
def _ffoverlay(X, y_true, embedding):
    x_flat = X.T.reshape(8, 8, 128, 128).transpose(0, 2, 1, 3).reshape(-1)
    mesh = plsc.VectorSubcoreMesh(core_axis_name="c", subcore_axis_name="s")
    run = pl.kernel(
        _body,
        out_type=jax.ShapeDtypeStruct((BATCH * DIM,), jnp.float32),
        mesh=mesh,
        scratch_types=[
            pltpu.VMEM((COLS_PER_W, 128), jnp.int32),
            pltpu.VMEM((COLS_PER_W, 128, DIM), jnp.float32),
            pltpu.VMEM((COLS_PER_W, COL_WORDS), jnp.float32),
            pltpu.SemaphoreType.DMA,
        ],
        compiler_params=pltpu.CompilerParams(
            use_tc_tiling_on_sc=False, needs_layout_passes=False
        ),
    )
    out_flat = run(x_flat, y_true, embedding)
    return (
        out_flat.reshape(8, 128, 8, 128)
        .transpose(1, 3, 0, 2)
        .reshape(BATCH, DIM)
    )


def kernel(X, y_true, embedding):
    return _ffoverlay(X, y_true.astype(jnp.int32), embedding)


# speedup vs baseline: 1.1526x; 1.1526x over previous
import jax
import jax.numpy as jnp
from jax import lax
from jax.experimental import pallas as pl
from jax.experimental.pallas import tpu as pltpu
from jax.experimental.pallas import tpu_sc as plsc

BATCH = 16384
VOCAB = 100000
DIM = 64
LANES = 16

NUM_CORES = 2
NUM_SUBCORES = 16
NW = NUM_CORES * NUM_SUBCORES
COLS_PER_W = (BATCH // 128) // NW
TILE_WORDS = 8 * 128
COL_WORDS = DIM * 128
PITCH = 130


def _body(x_hbm, idx_hbm, emb_hbm, out_hbm,
          idx_v, idxg_v, g2_v, gp_v, x_v, gsem, xsem, osem):
    wid = lax.axis_index("s") * NUM_CORES + lax.axis_index("c")
    lane_iota = lax.iota(jnp.int32, LANES)

    for k in range(COLS_PER_W):
        t1 = wid * COLS_PER_W + k
        pltpu.sync_copy(idx_hbm.at[pl.ds(t1 * 128, 128)], idx_v.at[k])
    for k in range(COLS_PER_W):
        for j in range(128 // LANES):
            sl = pl.ds(j * LANES, LANES)
            idxg_v[k, sl] = jax.lax.shift_right_logical(idx_v[k, sl], 1)

    xdescs = []
    for k in range(COLS_PER_W):
        t1 = wid * COLS_PER_W + k
        for t0 in range(8):
            xdescs.append(pltpu.async_copy(
                x_hbm.at[pl.ds((t0 * 128 + t1) * TILE_WORDS, TILE_WORDS)],
                x_v.at[k, pl.ds(t0 * TILE_WORDS, TILE_WORDS)],
                xsem,
            ))

    gdescs = [None] * COLS_PER_W
    for k in range(2):
        gdescs[k] = pltpu.async_copy(
            emb_hbm.at[idxg_v.at[k]], g2_v.at[k % 2], gsem)
    for d in xdescs:
        d.wait()

    odescs = []
    for k in range(COLS_PER_W):
        gdescs[k].wait()

        def repack(l, carry, k=k):
            base = l * PITCH
            for j in range(128 // LANES):
                chunk = g2_v[k % 2, l, pl.ds(j * LANES, LANES)]
                plsc.store_scatter(
                    gp_v, [lane_iota + (base + j * LANES)], chunk)
            return carry

        lax.fori_loop(0, 128, repack, 0)

        if k + 2 < COLS_PER_W:
            gdescs[k + 2] = pltpu.async_copy(
                emb_hbm.at[idxg_v.at[k + 2]], g2_v.at[k % 2], gsem)

        for lc in range(128 // LANES):
            half = (idx_v[k, pl.ds(lc * LANES, LANES)] & 1) * DIM
            rowbase = (lane_iota + lc * LANES) * PITCH + half

            def add_dim(d, carry, rowbase=rowbase, lc=lc, k=k):
                col = plsc.load_gather(gp_v, [rowbase + d])
                sl = pl.ds(d * 128 + lc * LANES, LANES)
                x_v[k, sl] = x_v[k, sl] + col
                return carry

            lax.fori_loop(0, DIM, add_dim, 0)

        t1 = wid * COLS_PER_W + k
        for t0 in range(8):
            odescs.append(pltpu.async_copy(
                x_v.at[k, pl.ds(t0 * TILE_WORDS, TILE_WORDS)],
                out_hbm.at[pl.ds((t0 * 128 + t1) * TILE_WORDS, TILE_WORDS)],
                osem,
            ))
    for d in odescs:
        d.wait()


@jax.jit
def _ffoverlay(X, y_true, embedding):
    x_flat = X.T.reshape(8, 8, 128, 128).transpose(0, 2, 1, 3).reshape(-1)
    emb2 = embedding.reshape(VOCAB // 2, 128)
    mesh = plsc.VectorSubcoreMesh(core_axis_name="c", subcore_axis_name="s")
    run = pl.kernel(
        _body,
        out_type=jax.ShapeDtypeStruct((BATCH * DIM,), jnp.float32),
        mesh=mesh,
        scratch_types=[
            pltpu.VMEM((COLS_PER_W, 128), jnp.int32),
            pltpu.VMEM((COLS_PER_W, 128), jnp.int32),
            pltpu.VMEM((2, 128, 128), jnp.float32),
            pltpu.VMEM((128 * PITCH,), jnp.float32),
            pltpu.VMEM((COLS_PER_W, COL_WORDS), jnp.float32),
            pltpu.SemaphoreType.DMA,
            pltpu.SemaphoreType.DMA,
            pltpu.SemaphoreType.DMA,
        ],
        compiler_params=pltpu.CompilerParams(
            use_tc_tiling_on_sc=False, needs_layout_passes=False
        ),
    )
    out_flat = run(x_flat, y_true, emb2)
    return (
        out_flat.reshape(8, 128, 8, 128)
        .transpose(1, 3, 0, 2)
        .reshape(BATCH, DIM)
    )


def kernel(X, y_true, embedding):
    return _ffoverlay(X, y_true.astype(jnp.int32), embedding)


# speedup vs baseline: 1.3050x; 1.1323x over previous
import jax
import jax.numpy as jnp
from jax import lax
from jax.experimental import pallas as pl
from jax.experimental.pallas import tpu as pltpu
from jax.experimental.pallas import tpu_sc as plsc

BATCH = 16384
VOCAB = 100000
DIM = 64
LANES = 16

NUM_CORES = 2
NUM_SUBCORES = 16
NW = NUM_CORES * NUM_SUBCORES
COLS_PER_W = (BATCH // 128) // NW
TILE_WORDS = 8 * 128
COL_WORDS = DIM * 128
PITCH = 130

TGROUPS = (VOCAB + 1023) // 1024
TROWS = TGROUPS * 512


def _retile_body(lo_ref, hi_ref, out_ref):
    out_ref[...] = jnp.concatenate(
        [lo_ref[...].T, hi_ref[...].T], axis=1)


def _retile(emb_t):
    return pl.pallas_call(
        _retile_body,
        grid=(TGROUPS,),
        in_specs=[
            pl.BlockSpec((64, 512), lambda j: (0, 2 * j)),
            pl.BlockSpec((64, 512), lambda j: (0, 2 * j + 1)),
        ],
        out_specs=pl.BlockSpec((512, 128), lambda j: (j, 0)),
        out_shape=jax.ShapeDtypeStruct((TROWS, 128), jnp.float32),
    )(emb_t, emb_t)


def _body(x_hbm, idx_hbm, emb_hbm, out_hbm,
          idx_v, idxg_v, g2_v, gp_v, x_v, gsem, xsem, osem):
    wid = lax.axis_index("s") * NUM_CORES + lax.axis_index("c")
    lane_iota = lax.iota(jnp.int32, LANES)

    for k in range(COLS_PER_W):
        t1 = wid * COLS_PER_W + k
        pltpu.sync_copy(idx_hbm.at[pl.ds(t1 * 128, 128)], idx_v.at[k])
    for k in range(COLS_PER_W):
        for j in range(128 // LANES):
            sl = pl.ds(j * LANES, LANES)
            v = idx_v[k, sl]
            idxg_v[k, sl] = (
                lax.shift_left(lax.shift_right_logical(v, 10), 9)
                | (v & 511)
            )

    xdescs = []
    for k in range(COLS_PER_W):
        t1 = wid * COLS_PER_W + k
        for t0 in range(8):
            xdescs.append(pltpu.async_copy(
                x_hbm.at[pl.ds((t0 * 128 + t1) * TILE_WORDS, TILE_WORDS)],
                x_v.at[k, pl.ds(t0 * TILE_WORDS, TILE_WORDS)],
                xsem,
            ))

    gdescs = [None] * COLS_PER_W
    for k in range(2):
        gdescs[k] = pltpu.async_copy(
            emb_hbm.at[idxg_v.at[k]], g2_v.at[k % 2], gsem)
    for d in xdescs:
        d.wait()

    odescs = []
    for k in range(COLS_PER_W):
        gdescs[k].wait()

        @plsc.parallel_loop(0, 128, unroll=2)
        def _repack(l, k=k):
            base = l * PITCH
            for j in range(128 // LANES):
                chunk = g2_v[k % 2, l, pl.ds(j * LANES, LANES)]
                plsc.store_scatter(
                    gp_v, [lane_iota + (base + j * LANES)], chunk)

        if k + 2 < COLS_PER_W:
            gdescs[k + 2] = pltpu.async_copy(
                emb_hbm.at[idxg_v.at[k + 2]], g2_v.at[k % 2], gsem)

        for lc in range(128 // LANES):
            v = idx_v[k, pl.ds(lc * LANES, LANES)]
            half = (lax.shift_right_logical(v, 9) & 1) * DIM
            rowbase = (lane_iota + lc * LANES) * PITCH + half

            @plsc.parallel_loop(0, DIM, unroll=2)
            def _add_dim(d, rowbase=rowbase, lc=lc, k=k):
                col = plsc.load_gather(gp_v, [rowbase + d])
                sl = pl.ds(d * 128 + lc * LANES, LANES)
                x_v[k, sl] = x_v[k, sl] + col

        t1 = wid * COLS_PER_W + k
        for t0 in range(8):
            odescs.append(pltpu.async_copy(
                x_v.at[k, pl.ds(t0 * TILE_WORDS, TILE_WORDS)],
                out_hbm.at[pl.ds((t0 * 128 + t1) * TILE_WORDS, TILE_WORDS)],
                osem,
            ))
    for d in odescs:
        d.wait()


@jax.jit
def _ffoverlay(X, y_true, embedding):
    x_flat = X.T.reshape(8, 8, 128, 128).transpose(0, 2, 1, 3).reshape(-1)
    table = _retile(embedding.T)
    mesh = plsc.VectorSubcoreMesh(core_axis_name="c", subcore_axis_name="s")
    run = pl.kernel(
        _body,
        out_type=jax.ShapeDtypeStruct((BATCH * DIM,), jnp.float32),
        mesh=mesh,
        scratch_types=[
            pltpu.VMEM((COLS_PER_W, 128), jnp.int32),
            pltpu.VMEM((COLS_PER_W, 128), jnp.int32),
            pltpu.VMEM((2, 128, 128), jnp.float32),
            pltpu.VMEM((128 * PITCH,), jnp.float32),
            pltpu.VMEM((COLS_PER_W, COL_WORDS), jnp.float32),
            pltpu.SemaphoreType.DMA,
            pltpu.SemaphoreType.DMA,
            pltpu.SemaphoreType.DMA,
        ],
        compiler_params=pltpu.CompilerParams(
            use_tc_tiling_on_sc=False, needs_layout_passes=False
        ),
    )
    out_flat = run(x_flat, y_true, table)
    return (
        out_flat.reshape(8, 128, 8, 128)
        .transpose(1, 3, 0, 2)
        .reshape(BATCH, DIM)
    )


def kernel(X, y_true, embedding):
    return _ffoverlay(X, y_true.astype(jnp.int32), embedding)


# speedup vs baseline: 1.7199x; 1.3179x over previous
import jax
import jax.numpy as jnp
from jax import lax
from jax.experimental import pallas as pl
from jax.experimental.pallas import tpu as pltpu
from jax.experimental.pallas import tpu_sc as plsc

BATCH = 16384
VOCAB = 100000
DIM = 64
LANES = 16

NUM_CORES = 2
NUM_SUBCORES = 16
NW = NUM_CORES * NUM_SUBCORES
COLS_PER_W = (BATCH // 128) // NW
TILE_WORDS = 8 * 128
COL_WORDS = DIM * 128
PITCH = 130

GROUP = 2048
HGROUP = GROUP // 2
TGROUPS = (VOCAB + GROUP - 1) // GROUP
TROWS = TGROUPS * HGROUP


def _retile_body(lo_ref, hi_ref, out_ref):
    out_ref[...] = jnp.concatenate(
        [lo_ref[...].T, hi_ref[...].T], axis=1)


def _retile(emb_t):
    return pl.pallas_call(
        _retile_body,
        grid=(TGROUPS,),
        in_specs=[
            pl.BlockSpec((64, HGROUP), lambda j: (0, 2 * j)),
            pl.BlockSpec((64, HGROUP), lambda j: (0, 2 * j + 1)),
        ],
        out_specs=pl.BlockSpec((HGROUP, 128), lambda j: (j, 0)),
        out_shape=jax.ShapeDtypeStruct((TROWS, 128), jnp.float32),
    )(emb_t, emb_t)


def _body(x_hbm, idx_hbm, emb_hbm, out_hbm,
          idx_v, idxg_v, g2_v, gp_v, x_v, gsem, xsem, osem):
    wid = lax.axis_index("s") * NUM_CORES + lax.axis_index("c")
    lane_iota = lax.iota(jnp.int32, LANES)

    for k in range(COLS_PER_W):
        t1 = wid * COLS_PER_W + k
        pltpu.sync_copy(idx_hbm.at[pl.ds(t1 * 128, 128)], idx_v.at[k])
    for k in range(COLS_PER_W):
        for j in range(128 // LANES):
            sl = pl.ds(j * LANES, LANES)
            v = idx_v[k, sl]
            idxg_v[k, sl] = (
                lax.shift_right_logical(v, 11) * HGROUP
                + (v & (HGROUP - 1))
            )

    xdescs = []
    for k in range(COLS_PER_W):
        t1 = wid * COLS_PER_W + k
        for t0 in range(8):
            xdescs.append(pltpu.async_copy(
                x_hbm.at[pl.ds((t0 * 128 + t1) * TILE_WORDS, TILE_WORDS)],
                x_v.at[k, pl.ds(t0 * TILE_WORDS, TILE_WORDS)],
                xsem,
            ))

    gdescs = [None] * COLS_PER_W
    for k in range(2):
        gdescs[k] = pltpu.async_copy(
            emb_hbm.at[idxg_v.at[k]], g2_v.at[k % 2], gsem)
    for d in xdescs:
        d.wait()

    odescs = []
    for k in range(COLS_PER_W):
        gdescs[k].wait()

        @plsc.parallel_loop(0, 128, unroll=2)
        def _repack(l, k=k):
            base = l * PITCH
            for j in range(128 // LANES):
                chunk = g2_v[k % 2, l, pl.ds(j * LANES, LANES)]
                plsc.store_scatter(
                    gp_v, [lane_iota + (base + j * LANES)], chunk)

        if k + 2 < COLS_PER_W:
            gdescs[k + 2] = pltpu.async_copy(
                emb_hbm.at[idxg_v.at[k + 2]], g2_v.at[k % 2], gsem)

        for lc in range(128 // LANES):
            v = idx_v[k, pl.ds(lc * LANES, LANES)]
            half = (lax.shift_right_logical(v, 10) & 1) * DIM
            rowbase = (lane_iota + lc * LANES) * PITCH + half

            @plsc.parallel_loop(0, DIM, unroll=2)
            def _add_dim(d, rowbase=rowbase, lc=lc, k=k):
                col = plsc.load_gather(gp_v, [rowbase + d])
                sl = pl.ds(d * 128 + lc * LANES, LANES)
                x_v[k, sl] = x_v[k, sl] + col

        t1 = wid * COLS_PER_W + k
        for t0 in range(8):
            odescs.append(pltpu.async_copy(
                x_v.at[k, pl.ds(t0 * TILE_WORDS, TILE_WORDS)],
                out_hbm.at[pl.ds((t0 * 128 + t1) * TILE_WORDS, TILE_WORDS)],
                osem,
            ))
    for d in odescs:
        d.wait()


@jax.jit
def _ffoverlay(X, y_true, embedding):
    x_flat = X.T.reshape(8, 8, 128, 128).transpose(0, 2, 1, 3).reshape(-1)
    table = _retile(embedding.T)
    mesh = plsc.VectorSubcoreMesh(core_axis_name="c", subcore_axis_name="s")
    run = pl.kernel(
        _body,
        out_type=jax.ShapeDtypeStruct((BATCH * DIM,), jnp.float32),
        mesh=mesh,
        scratch_types=[
            pltpu.VMEM((COLS_PER_W, 128), jnp.int32),
            pltpu.VMEM((COLS_PER_W, 128), jnp.int32),
            pltpu.VMEM((2, 128, 128), jnp.float32),
            pltpu.VMEM((128 * PITCH,), jnp.float32),
            pltpu.VMEM((COLS_PER_W, COL_WORDS), jnp.float32),
            pltpu.SemaphoreType.DMA,
            pltpu.SemaphoreType.DMA,
            pltpu.SemaphoreType.DMA,
        ],
        compiler_params=pltpu.CompilerParams(
            use_tc_tiling_on_sc=False, needs_layout_passes=False
        ),
    )
    out_flat = run(x_flat, y_true, table)
    return (
        out_flat.reshape(8, 128, 8, 128)
        .transpose(1, 3, 0, 2)
        .reshape(BATCH, DIM)
    )


def kernel(X, y_true, embedding):
    return _ffoverlay(X, y_true.astype(jnp.int32), embedding)


# speedup vs baseline: 1.9998x; 1.1627x over previous
import jax
import jax.numpy as jnp
from jax import lax
from jax.experimental import pallas as pl
from jax.experimental.pallas import tpu as pltpu
from jax.experimental.pallas import tpu_sc as plsc

BATCH = 16384
VOCAB = 100000
DIM = 64
LANES = 16

NUM_CORES = 2
NUM_SUBCORES = 16
NW = NUM_CORES * NUM_SUBCORES
COLS_PER_W = (BATCH // 128) // NW
TILE_WORDS = 8 * 128
COL_WORDS = DIM * 128
PITCH = 130

GROUP = 2048
HGROUP = GROUP // 2
TGROUPS = (VOCAB + GROUP - 1) // GROUP
TROWS = TGROUPS * HGROUP


def _retile_body(lo1_ref, hi1_ref, lo2_ref, hi2_ref, out_ref):
    g1 = jnp.concatenate([lo1_ref[...].T, hi1_ref[...].T], axis=1)
    g2 = jnp.concatenate([lo2_ref[...].T, hi2_ref[...].T], axis=1)
    out_ref[...] = jnp.concatenate([g1, g2], axis=0)


def _retile(emb_t):
    nsteps = (TGROUPS + 1) // 2
    return pl.pallas_call(
        _retile_body,
        grid=(nsteps,),
        in_specs=[
            pl.BlockSpec((64, HGROUP),
                         lambda j: (0, jnp.minimum(4 * j, 2 * TGROUPS - 1))),
            pl.BlockSpec((64, HGROUP),
                         lambda j: (0, jnp.minimum(4 * j + 1,
                                                   2 * TGROUPS - 1))),
            pl.BlockSpec((64, HGROUP),
                         lambda j: (0, jnp.minimum(4 * j + 2,
                                                   2 * TGROUPS - 1))),
            pl.BlockSpec((64, HGROUP),
                         lambda j: (0, jnp.minimum(4 * j + 3,
                                                   2 * TGROUPS - 1))),
        ],
        out_specs=pl.BlockSpec((2 * HGROUP, 128), lambda j: (j, 0)),
        out_shape=jax.ShapeDtypeStruct((2 * HGROUP * nsteps, 128),
                                       jnp.float32),
    )(emb_t, emb_t, emb_t, emb_t)


def _body(x_hbm, idx_hbm, emb_hbm, out_hbm,
          idx_v, idxg_v, g2_v, gp_v, x_v, gsem, xsem, osem):
    wid = lax.axis_index("s") * NUM_CORES + lax.axis_index("c")
    lane_iota = lax.iota(jnp.int32, LANES)

    for k in range(COLS_PER_W):
        t1 = wid * COLS_PER_W + k
        pltpu.sync_copy(idx_hbm.at[pl.ds(t1 * 128, 128)], idx_v.at[k])
    for k in range(COLS_PER_W):
        for j in range(128 // LANES):
            sl = pl.ds(j * LANES, LANES)
            v = idx_v[k, sl]
            idxg_v[k, sl] = (
                lax.shift_right_logical(v, 11) * HGROUP
                + (v & (HGROUP - 1))
            )

    xdescs = []
    for k in range(COLS_PER_W):
        t1 = wid * COLS_PER_W + k
        for t0 in range(8):
            xdescs.append(pltpu.async_copy(
                x_hbm.at[pl.ds((t0 * 128 + t1) * TILE_WORDS, TILE_WORDS)],
                x_v.at[k, pl.ds(t0 * TILE_WORDS, TILE_WORDS)],
                xsem,
            ))

    gdescs = [None] * COLS_PER_W
    for k in range(2):
        gdescs[k] = pltpu.async_copy(
            emb_hbm.at[idxg_v.at[k]], g2_v.at[k % 2], gsem)
    for d in xdescs:
        d.wait()

    odescs = []
    for k in range(COLS_PER_W):
        gdescs[k].wait()

        @plsc.parallel_loop(0, 128, unroll=2)
        def _repack(l, k=k):
            base = l * PITCH
            for j in range(128 // LANES):
                chunk = g2_v[k % 2, l, pl.ds(j * LANES, LANES)]
                plsc.store_scatter(
                    gp_v, [lane_iota + (base + j * LANES)], chunk)

        if k + 2 < COLS_PER_W:
            gdescs[k + 2] = pltpu.async_copy(
                emb_hbm.at[idxg_v.at[k + 2]], g2_v.at[k % 2], gsem)

        for lc in range(128 // LANES):
            v = idx_v[k, pl.ds(lc * LANES, LANES)]
            half = (lax.shift_right_logical(v, 10) & 1) * DIM
            rowbase = (lane_iota + lc * LANES) * PITCH + half

            @plsc.parallel_loop(0, DIM, unroll=2)
            def _add_dim(d, rowbase=rowbase, lc=lc, k=k):
                col = plsc.load_gather(gp_v, [rowbase + d])
                sl = pl.ds(d * 128 + lc * LANES, LANES)
                x_v[k, sl] = x_v[k, sl] + col

        t1 = wid * COLS_PER_W + k
        for t0 in range(8):
            odescs.append(pltpu.async_copy(
                x_v.at[k, pl.ds(t0 * TILE_WORDS, TILE_WORDS)],
                out_hbm.at[pl.ds((t0 * 128 + t1) * TILE_WORDS, TILE_WORDS)],
                osem,
            ))
    for d in odescs:
        d.wait()


@jax.jit
def _ffoverlay(X, y_true, embedding):
    x_flat = X.T.reshape(8, 8, 128, 128).transpose(0, 2, 1, 3).reshape(-1)
    table = _retile(embedding.T)
    mesh = plsc.VectorSubcoreMesh(core_axis_name="c", subcore_axis_name="s")
    run = pl.kernel(
        _body,
        out_type=jax.ShapeDtypeStruct((BATCH * DIM,), jnp.float32),
        mesh=mesh,
        scratch_types=[
            pltpu.VMEM((COLS_PER_W, 128), jnp.int32),
            pltpu.VMEM((COLS_PER_W, 128), jnp.int32),
            pltpu.VMEM((2, 128, 128), jnp.float32),
            pltpu.VMEM((128 * PITCH,), jnp.float32),
            pltpu.VMEM((COLS_PER_W, COL_WORDS), jnp.float32),
            pltpu.SemaphoreType.DMA,
            pltpu.SemaphoreType.DMA,
            pltpu.SemaphoreType.DMA,
        ],
        compiler_params=pltpu.CompilerParams(
            use_tc_tiling_on_sc=False, needs_layout_passes=False
        ),
    )
    out_flat = run(x_flat, y_true, table)
    return (
        out_flat.reshape(8, 128, 8, 128)
        .transpose(1, 3, 0, 2)
        .reshape(BATCH, DIM)
    )


def kernel(X, y_true, embedding):
    return _ffoverlay(X, y_true.astype(jnp.int32), embedding)


# speedup vs baseline: 2.1507x; 1.0755x over previous
import jax
import jax.numpy as jnp
from jax import lax
from jax.experimental import pallas as pl
from jax.experimental.pallas import tpu as pltpu
from jax.experimental.pallas import tpu_sc as plsc

BATCH = 16384
VOCAB = 100000
DIM = 64
LANES = 16

NUM_CORES = 2
NUM_SUBCORES = 16
NW = NUM_CORES * NUM_SUBCORES
COLS_PER_W = (BATCH // 128) // NW
TILE_WORDS = 8 * 128
COL_WORDS = DIM * 128
PITCH = 130

GROUP = 2048
HGROUP = GROUP // 2
TGROUPS = (VOCAB + GROUP - 1) // GROUP
TROWS = TGROUPS * HGROUP


GPS = 4


def _retile_body(*refs):
    out_ref = refs[-1]
    parts = [
        jnp.concatenate(
            [refs[2 * i][...].T, refs[2 * i + 1][...].T], axis=1)
        for i in range(GPS)
    ]
    out_ref[...] = jnp.concatenate(parts, axis=0)


def _retile(emb_t):
    nsteps = (TGROUPS + GPS - 1) // GPS
    lim = 2 * TGROUPS - 1

    def _mk(i):
        return pl.BlockSpec(
            (64, HGROUP),
            lambda j: (0, jnp.minimum(2 * GPS * j + i, lim)))

    return pl.pallas_call(
        _retile_body,
        grid=(nsteps,),
        in_specs=[_mk(i) for i in range(2 * GPS)],
        out_specs=pl.BlockSpec((GPS * HGROUP, 128), lambda j: (j, 0)),
        out_shape=jax.ShapeDtypeStruct((GPS * HGROUP * nsteps, 128),
                                       jnp.float32),
    )(*([emb_t] * (2 * GPS)))


def _body(x_hbm, idx_hbm, emb_hbm, out_hbm,
          idx_v, idxg_v, g2_v, gp_v, x_v, gsem, xsem, osem):
    wid = lax.axis_index("s") * NUM_CORES + lax.axis_index("c")
    lane_iota = lax.iota(jnp.int32, LANES)

    for k in range(COLS_PER_W):
        t1 = wid * COLS_PER_W + k
        pltpu.sync_copy(idx_hbm.at[pl.ds(t1 * 128, 128)], idx_v.at[k])
    for k in range(COLS_PER_W):
        for j in range(128 // LANES):
            sl = pl.ds(j * LANES, LANES)
            v = idx_v[k, sl]
            idxg_v[k, sl] = (
                lax.shift_right_logical(v, 11) * HGROUP
                + (v & (HGROUP - 1))
            )

    xdescs = []
    for k in range(COLS_PER_W):
        t1 = wid * COLS_PER_W + k
        for t0 in range(8):
            xdescs.append(pltpu.async_copy(
                x_hbm.at[pl.ds((t0 * 128 + t1) * TILE_WORDS, TILE_WORDS)],
                x_v.at[k, pl.ds(t0 * TILE_WORDS, TILE_WORDS)],
                xsem,
            ))

    gdescs = [None] * COLS_PER_W
    for k in range(2):
        gdescs[k] = pltpu.async_copy(
            emb_hbm.at[idxg_v.at[k]], g2_v.at[k % 2], gsem)
    for d in xdescs:
        d.wait()

    odescs = []
    for k in range(COLS_PER_W):
        gdescs[k].wait()

        @plsc.parallel_loop(0, 128, unroll=2)
        def _repack(l, k=k):
            base = l * PITCH
            for j in range(128 // LANES):
                chunk = g2_v[k % 2, l, pl.ds(j * LANES, LANES)]
                plsc.store_scatter(
                    gp_v, [lane_iota + (base + j * LANES)], chunk)

        if k + 2 < COLS_PER_W:
            gdescs[k + 2] = pltpu.async_copy(
                emb_hbm.at[idxg_v.at[k + 2]], g2_v.at[k % 2], gsem)

        for lc in range(128 // LANES):
            v = idx_v[k, pl.ds(lc * LANES, LANES)]
            half = (lax.shift_right_logical(v, 10) & 1) * DIM
            rowbase = (lane_iota + lc * LANES) * PITCH + half

            @plsc.parallel_loop(0, DIM, unroll=2)
            def _add_dim(d, rowbase=rowbase, lc=lc, k=k):
                col = plsc.load_gather(gp_v, [rowbase + d])
                sl = pl.ds(d * 128 + lc * LANES, LANES)
                x_v[k, sl] = x_v[k, sl] + col

        t1 = wid * COLS_PER_W + k
        for t0 in range(8):
            odescs.append(pltpu.async_copy(
                x_v.at[k, pl.ds(t0 * TILE_WORDS, TILE_WORDS)],
                out_hbm.at[pl.ds((t0 * 128 + t1) * TILE_WORDS, TILE_WORDS)],
                osem,
            ))
    for d in odescs:
        d.wait()


@jax.jit
def _ffoverlay(X, y_true, embedding):
    x_flat = X.T.reshape(8, 8, 128, 128).transpose(0, 2, 1, 3).reshape(-1)
    table = _retile(embedding.T)
    mesh = plsc.VectorSubcoreMesh(core_axis_name="c", subcore_axis_name="s")
    run = pl.kernel(
        _body,
        out_type=jax.ShapeDtypeStruct((BATCH * DIM,), jnp.float32),
        mesh=mesh,
        scratch_types=[
            pltpu.VMEM((COLS_PER_W, 128), jnp.int32),
            pltpu.VMEM((COLS_PER_W, 128), jnp.int32),
            pltpu.VMEM((2, 128, 128), jnp.float32),
            pltpu.VMEM((128 * PITCH,), jnp.float32),
            pltpu.VMEM((COLS_PER_W, COL_WORDS), jnp.float32),
            pltpu.SemaphoreType.DMA,
            pltpu.SemaphoreType.DMA,
            pltpu.SemaphoreType.DMA,
        ],
        compiler_params=pltpu.CompilerParams(
            use_tc_tiling_on_sc=False, needs_layout_passes=False
        ),
    )
    out_flat = run(x_flat, y_true, table)
    return (
        out_flat.reshape(8, 128, 8, 128)
        .transpose(1, 3, 0, 2)
        .reshape(BATCH, DIM)
    )


def kernel(X, y_true, embedding):
    return _ffoverlay(X, y_true.astype(jnp.int32), embedding)


# speedup vs baseline: 2.1846x; 1.0157x over previous
import jax
import jax.numpy as jnp
from jax import lax
from jax.experimental import pallas as pl
from jax.experimental.pallas import tpu as pltpu
from jax.experimental.pallas import tpu_sc as plsc

BATCH = 16384
VOCAB = 100000
DIM = 64
LANES = 16

NUM_CORES = 2
NUM_SUBCORES = 16
NW = NUM_CORES * NUM_SUBCORES
COLS_PER_W = (BATCH // 128) // NW
TILE_WORDS = 8 * 128
COL_WORDS = DIM * 128
PITCH = 65

GROUP = 2048
HGROUP = GROUP // 2
TGROUPS = (VOCAB + GROUP - 1) // GROUP
TROWS = TGROUPS * HGROUP


GPS = 8


def _retile_body(*refs):
    out_ref = refs[-1]
    parts = [
        jnp.concatenate(
            [refs[2 * i][...].T, refs[2 * i + 1][...].T], axis=1)
        for i in range(GPS)
    ]
    out_ref[...] = jnp.concatenate(parts, axis=0)


def _retile(emb_t):
    nsteps = (TGROUPS + GPS - 1) // GPS
    lim = 2 * TGROUPS - 1

    def _mk(i):
        return pl.BlockSpec(
            (64, HGROUP),
            lambda j: (0, jnp.minimum(2 * GPS * j + i, lim)))

    return pl.pallas_call(
        _retile_body,
        grid=(nsteps,),
        in_specs=[_mk(i) for i in range(2 * GPS)],
        out_specs=pl.BlockSpec((GPS * HGROUP, 128), lambda j: (j, 0)),
        out_shape=jax.ShapeDtypeStruct((GPS * HGROUP * nsteps, 128),
                                       jnp.float32),
    )(*([emb_t] * (2 * GPS)))


def _body(x_hbm, idx_hbm, emb_hbm, out_hbm,
          idx_v, idxg_v, g2_v, gp_v, x_v, gsem, xsem, osem):
    wid = lax.axis_index("s") * NUM_CORES + lax.axis_index("c")
    lane_iota = lax.iota(jnp.int32, LANES)

    for k in range(COLS_PER_W):
        t1 = wid * COLS_PER_W + k
        pltpu.sync_copy(idx_hbm.at[pl.ds(t1 * 128, 128)], idx_v.at[k])
    for k in range(COLS_PER_W):
        for j in range(128 // LANES):
            sl = pl.ds(j * LANES, LANES)
            v = idx_v[k, sl]
            idxg_v[k, sl] = (
                lax.shift_right_logical(v, 11) * (2 * HGROUP)
                + (v & (HGROUP - 1)) * 2
                + (lax.shift_right_logical(v, 10) & 1)
            )

    xdescs = []
    for k in range(COLS_PER_W):
        t1 = wid * COLS_PER_W + k
        for t0 in range(8):
            xdescs.append(pltpu.async_copy(
                x_hbm.at[pl.ds((t0 * 128 + t1) * TILE_WORDS, TILE_WORDS)],
                x_v.at[k, pl.ds(t0 * TILE_WORDS, TILE_WORDS)],
                xsem,
            ))

    gdescs = [None] * COLS_PER_W
    for k in range(2):
        gdescs[k] = pltpu.async_copy(
            emb_hbm.at[idxg_v.at[k]], g2_v.at[k % 2], gsem)
    for d in xdescs:
        d.wait()

    odescs = []
    for k in range(COLS_PER_W):
        gdescs[k].wait()

        @plsc.parallel_loop(0, 128, unroll=2)
        def _repack(l, k=k):
            base = l * PITCH
            for j in range(DIM // LANES):
                chunk = g2_v[k % 2, l, pl.ds(j * LANES, LANES)]
                plsc.store_scatter(
                    gp_v, [lane_iota + (base + j * LANES)], chunk)

        if k + 2 < COLS_PER_W:
            gdescs[k + 2] = pltpu.async_copy(
                emb_hbm.at[idxg_v.at[k + 2]], g2_v.at[k % 2], gsem)

        for lc in range(128 // LANES):
            rowbase = (lane_iota + lc * LANES) * PITCH

            @plsc.parallel_loop(0, DIM, unroll=2)
            def _add_dim(d, rowbase=rowbase, lc=lc, k=k):
                col = plsc.load_gather(gp_v, [rowbase + d])
                sl = pl.ds(d * 128 + lc * LANES, LANES)
                x_v[k, sl] = x_v[k, sl] + col

        t1 = wid * COLS_PER_W + k
        for t0 in range(8):
            odescs.append(pltpu.async_copy(
                x_v.at[k, pl.ds(t0 * TILE_WORDS, TILE_WORDS)],
                out_hbm.at[pl.ds((t0 * 128 + t1) * TILE_WORDS, TILE_WORDS)],
                osem,
            ))
    for d in odescs:
        d.wait()


@jax.jit
def _ffoverlay(X, y_true, embedding):
    x_flat = X.T.reshape(8, 8, 128, 128).transpose(0, 2, 1, 3).reshape(-1)
    table = _retile(embedding.T).reshape(-1, DIM)
    mesh = plsc.VectorSubcoreMesh(core_axis_name="c", subcore_axis_name="s")
    run = pl.kernel(
        _body,
        out_type=jax.ShapeDtypeStruct((BATCH * DIM,), jnp.float32),
        mesh=mesh,
        scratch_types=[
            pltpu.VMEM((COLS_PER_W, 128), jnp.int32),
            pltpu.VMEM((COLS_PER_W, 128), jnp.int32),
            pltpu.VMEM((2, 128, DIM), jnp.float32),
            pltpu.VMEM((128 * PITCH,), jnp.float32),
            pltpu.VMEM((COLS_PER_W, COL_WORDS), jnp.float32),
            pltpu.SemaphoreType.DMA,
            pltpu.SemaphoreType.DMA,
            pltpu.SemaphoreType.DMA,
        ],
        compiler_params=pltpu.CompilerParams(
            use_tc_tiling_on_sc=False, needs_layout_passes=False
        ),
    )
    out_flat = run(x_flat, y_true, table)
    return (
        out_flat.reshape(8, 128, 8, 128)
        .transpose(1, 3, 0, 2)
        .reshape(BATCH, DIM)
    )


def kernel(X, y_true, embedding):
    return _ffoverlay(X, y_true.astype(jnp.int32), embedding)


# speedup vs baseline: 2.2001x; 1.0071x over previous
import jax
import jax.numpy as jnp
from jax import lax
from jax.experimental import pallas as pl
from jax.experimental.pallas import tpu as pltpu
from jax.experimental.pallas import tpu_sc as plsc

BATCH = 16384
VOCAB = 100000
DIM = 64
LANES = 16

NUM_CORES = 2
NUM_SUBCORES = 16
NW = NUM_CORES * NUM_SUBCORES
COLS_PER_W = (BATCH // 128) // NW
TILE_WORDS = 8 * 128
COL_WORDS = DIM * 128
PITCH = 65

GROUP = 2048
HGROUP = GROUP // 2
TGROUPS = (VOCAB + GROUP - 1) // GROUP
TROWS = TGROUPS * HGROUP


GPS = 8


def _retile_body(*refs):
    out_ref = refs[-1]
    parts = [
        jnp.concatenate(
            [refs[2 * i][...].T, refs[2 * i + 1][...].T], axis=1)
        for i in range(GPS)
    ]
    out_ref[...] = jnp.concatenate(parts, axis=0)


def _retile(emb_t):
    nsteps = (TGROUPS + GPS - 1) // GPS
    lim = 2 * TGROUPS - 1

    def _mk(i):
        return pl.BlockSpec(
            (64, HGROUP),
            lambda j: (0, jnp.minimum(2 * GPS * j + i, lim)))

    return pl.pallas_call(
        _retile_body,
        grid=(nsteps,),
        in_specs=[_mk(i) for i in range(2 * GPS)],
        out_specs=pl.BlockSpec((GPS * HGROUP, 128), lambda j: (j, 0)),
        out_shape=jax.ShapeDtypeStruct((GPS * HGROUP * nsteps, 128),
                                       jnp.float32),
    )(*([emb_t] * (2 * GPS)))


def _body(x_hbm, idx_hbm, emb_hbm, out_hbm,
          idx_v, idxg_v, g2_v, gp_v, x_v, gsem, xsem, osem):
    wid = lax.axis_index("s") * NUM_CORES + lax.axis_index("c")
    lane_iota = lax.iota(jnp.int32, LANES)

    for k in range(COLS_PER_W):
        t1 = wid * COLS_PER_W + k
        pltpu.sync_copy(idx_hbm.at[pl.ds(t1 * 128, 128)], idx_v.at[k])
    for k in range(COLS_PER_W):
        for j in range(128 // LANES):
            sl = pl.ds(j * LANES, LANES)
            v = idx_v[k, sl]
            idxg_v[k, sl] = (
                lax.shift_right_logical(v, 11) * (2 * HGROUP)
                + (v & (HGROUP - 1)) * 2
                + (lax.shift_right_logical(v, 10) & 1)
            )

    xdescs = []
    for k in range(COLS_PER_W):
        t1 = wid * COLS_PER_W + k
        for t0 in range(8):
            xdescs.append(pltpu.async_copy(
                x_hbm.at[pl.ds((t0 * 128 + t1) * TILE_WORDS, TILE_WORDS)],
                x_v.at[k, pl.ds(t0 * TILE_WORDS, TILE_WORDS)],
                xsem,
            ))

    gdescs = [None] * COLS_PER_W
    for k in range(2):
        gdescs[k] = pltpu.async_copy(
            emb_hbm.at[idxg_v.at[k]], g2_v.at[k % 2], gsem)
    for d in xdescs:
        d.wait()

    odescs = []
    for k in range(COLS_PER_W):
        gdescs[k].wait()

        @plsc.parallel_loop(0, 128, unroll=4)
        def _repack(l, k=k):
            base = l * PITCH
            for j in range(DIM // LANES):
                chunk = g2_v[k % 2, l, pl.ds(j * LANES, LANES)]
                plsc.store_scatter(
                    gp_v, [lane_iota + (base + j * LANES)], chunk)

        if k + 2 < COLS_PER_W:
            gdescs[k + 2] = pltpu.async_copy(
                emb_hbm.at[idxg_v.at[k + 2]], g2_v.at[k % 2], gsem)

        for lc in range(128 // LANES):
            rowbase = (lane_iota + lc * LANES) * PITCH

            @plsc.parallel_loop(0, DIM, unroll=4)
            def _add_dim(d, rowbase=rowbase, lc=lc, k=k):
                col = plsc.load_gather(gp_v, [rowbase + d])
                sl = pl.ds(d * 128 + lc * LANES, LANES)
                x_v[k, sl] = x_v[k, sl] + col

        t1 = wid * COLS_PER_W + k
        for t0 in range(8):
            odescs.append(pltpu.async_copy(
                x_v.at[k, pl.ds(t0 * TILE_WORDS, TILE_WORDS)],
                out_hbm.at[pl.ds((t0 * 128 + t1) * TILE_WORDS, TILE_WORDS)],
                osem,
            ))
    for d in odescs:
        d.wait()


@jax.jit
def _ffoverlay(X, y_true, embedding):
    x_flat = X.T.reshape(8, 8, 128, 128).transpose(0, 2, 1, 3).reshape(-1)
    table = _retile(embedding.T).reshape(-1, DIM)
    mesh = plsc.VectorSubcoreMesh(core_axis_name="c", subcore_axis_name="s")
    run = pl.kernel(
        _body,
        out_type=jax.ShapeDtypeStruct((BATCH * DIM,), jnp.float32),
        mesh=mesh,
        scratch_types=[
            pltpu.VMEM((COLS_PER_W, 128), jnp.int32),
            pltpu.VMEM((COLS_PER_W, 128), jnp.int32),
            pltpu.VMEM((2, 128, DIM), jnp.float32),
            pltpu.VMEM((128 * PITCH,), jnp.float32),
            pltpu.VMEM((COLS_PER_W, COL_WORDS), jnp.float32),
            pltpu.SemaphoreType.DMA,
            pltpu.SemaphoreType.DMA,
            pltpu.SemaphoreType.DMA,
        ],
        compiler_params=pltpu.CompilerParams(
            use_tc_tiling_on_sc=False, needs_layout_passes=False
        ),
    )
    out_flat = run(x_flat, y_true, table)
    return (
        out_flat.reshape(8, 128, 8, 128)
        .transpose(1, 3, 0, 2)
        .reshape(BATCH, DIM)
    )


def kernel(X, y_true, embedding):
    return _ffoverlay(X, y_true.astype(jnp.int32), embedding)


# speedup vs baseline: 2.2631x; 1.0286x over previous
import jax
import jax.numpy as jnp
from jax import lax
from jax.experimental import pallas as pl
from jax.experimental.pallas import tpu as pltpu
from jax.experimental.pallas import tpu_sc as plsc

BATCH = 16384
VOCAB = 100000
DIM = 64
LANES = 16

NUM_CORES = 2
NUM_SUBCORES = 16
NW = NUM_CORES * NUM_SUBCORES
COLS_PER_W = (BATCH // 128) // NW
TILE_WORDS = 8 * 128
COL_WORDS = DIM * 128
PITCH = 65

GROUP = 2048
HGROUP = GROUP // 2
TGROUPS = (VOCAB + GROUP - 1) // GROUP
TROWS = TGROUPS * HGROUP


GPS = 4


def _retile_body(*refs):
    out_ref = refs[-1]
    parts = [
        jnp.concatenate(
            [refs[2 * i][...].T, refs[2 * i + 1][...].T], axis=1)
        for i in range(GPS)
    ]
    out_ref[...] = jnp.concatenate(parts, axis=0)


def _retile(emb_t):
    nsteps = (TGROUPS + GPS - 1) // GPS
    lim = 2 * TGROUPS - 1

    def _mk(i):
        return pl.BlockSpec(
            (64, HGROUP),
            lambda j: (0, jnp.minimum(2 * GPS * j + i, lim)))

    return pl.pallas_call(
        _retile_body,
        grid=(nsteps,),
        in_specs=[_mk(i) for i in range(2 * GPS)],
        out_specs=pl.BlockSpec((GPS * HGROUP, 128), lambda j: (j, 0)),
        out_shape=jax.ShapeDtypeStruct((GPS * HGROUP * nsteps, 128),
                                       jnp.float32),
    )(*([emb_t] * (2 * GPS)))


def _body(x_hbm, idx_hbm, emb_hbm, out_hbm,
          idx_v, idxg_v, g2_v, gp_v, x_v, gsem, xsem, osem):
    wid = lax.axis_index("s") * NUM_CORES + lax.axis_index("c")
    lane_iota = lax.iota(jnp.int32, LANES)

    for k in range(COLS_PER_W):
        t1 = wid * COLS_PER_W + k
        pltpu.sync_copy(idx_hbm.at[pl.ds(t1 * 128, 128)], idx_v.at[k])
    for k in range(COLS_PER_W):
        for j in range(128 // LANES):
            sl = pl.ds(j * LANES, LANES)
            v = idx_v[k, sl]
            idxg_v[k, sl] = (
                lax.shift_right_logical(v, 11) * (2 * HGROUP)
                + (v & (HGROUP - 1)) * 2
                + (lax.shift_right_logical(v, 10) & 1)
            )

    xdescs = []
    for k in range(COLS_PER_W):
        t1 = wid * COLS_PER_W + k
        for t0 in range(8):
            xdescs.append(pltpu.async_copy(
                x_hbm.at[pl.ds((t0 * 128 + t1) * TILE_WORDS, TILE_WORDS)],
                x_v.at[k, pl.ds(t0 * TILE_WORDS, TILE_WORDS)],
                xsem,
            ))

    gdescs = [None] * COLS_PER_W
    for k in range(2):
        gdescs[k] = pltpu.async_copy(
            emb_hbm.at[idxg_v.at[k]], g2_v.at[k % 2], gsem)
    for d in xdescs:
        d.wait()

    odescs = []
    for k in range(COLS_PER_W):
        gdescs[k].wait()

        @plsc.parallel_loop(0, 128, unroll=4)
        def _repack(l, k=k):
            base = l * PITCH
            for j in range(DIM // LANES):
                chunk = g2_v[k % 2, l, pl.ds(j * LANES, LANES)]
                plsc.store_scatter(
                    gp_v, [lane_iota + (base + j * LANES)], chunk)

        if k + 2 < COLS_PER_W:
            gdescs[k + 2] = pltpu.async_copy(
                emb_hbm.at[idxg_v.at[k + 2]], g2_v.at[k % 2], gsem)

        for lc in range(128 // LANES):
            rowbase = (lane_iota + lc * LANES) * PITCH

            @plsc.parallel_loop(0, DIM, unroll=4)
            def _add_dim(d, rowbase=rowbase, lc=lc, k=k):
                col = plsc.load_gather(gp_v, [rowbase + d])
                sl = pl.ds(d * 128 + lc * LANES, LANES)
                x_v[k, sl] = x_v[k, sl] + col

        t1 = wid * COLS_PER_W + k
        for t0 in range(8):
            odescs.append(pltpu.async_copy(
                x_v.at[k, pl.ds(t0 * TILE_WORDS, TILE_WORDS)],
                out_hbm.at[pl.ds((t0 * 128 + t1) * TILE_WORDS, TILE_WORDS)],
                osem,
            ))
    for d in odescs:
        d.wait()


@jax.jit
def _ffoverlay(X, y_true, embedding):
    x_flat = X.T.reshape(8, 8, 128, 128).transpose(0, 2, 1, 3).reshape(-1)
    table = _retile(embedding.T).reshape(-1, DIM)
    mesh = plsc.VectorSubcoreMesh(core_axis_name="c", subcore_axis_name="s")
    run = pl.kernel(
        _body,
        out_type=jax.ShapeDtypeStruct((BATCH * DIM,), jnp.float32),
        mesh=mesh,
        scratch_types=[
            pltpu.VMEM((COLS_PER_W, 128), jnp.int32),
            pltpu.VMEM((COLS_PER_W, 128), jnp.int32),
            pltpu.VMEM((2, 128, DIM), jnp.float32),
            pltpu.VMEM((128 * PITCH,), jnp.float32),
            pltpu.VMEM((COLS_PER_W, COL_WORDS), jnp.float32),
            pltpu.SemaphoreType.DMA,
            pltpu.SemaphoreType.DMA,
            pltpu.SemaphoreType.DMA,
        ],
        compiler_params=pltpu.CompilerParams(
            use_tc_tiling_on_sc=False, needs_layout_passes=False
        ),
    )
    out_flat = run(x_flat, y_true, table)
    return (
        out_flat.reshape(8, 128, 8, 128)
        .transpose(1, 3, 0, 2)
        .reshape(BATCH, DIM)
    )


def kernel(X, y_true, embedding):
    return _ffoverlay(X, y_true.astype(jnp.int32), embedding)
